# Initial kernel scaffold; baseline (speedup 1.0000x reference)
import jax
import jax.numpy as jnp
from jax.experimental import pallas as pl


def kernel(x, z, H, w, psi_W, psi_b, phi_W, phi_b, g1_W, g1_b, g2_W, g2_b, c1_W, c1_b, c2_W, c2_b, hd_W, hd_b):
    raise NotImplementedError("write your pallas kernel here")


# speedup vs baseline: 1.0347x; 1.0347x over previous
import jax
import jax.numpy as jnp
from jax.experimental import pallas as pl
from jax.experimental.pallas import tpu as pltpu

N = 10000
M = 2048
BN = 1000
EPS = 1e-9


def _pass1_kernel(h_ref, x_ref, z_ref, w_ref,
                  psi_W_ref, psi_b_ref, phi_W_ref, phi_b_ref,
                  g1_W_ref, g1_b_ref, g2_W_ref, g2_b_ref,
                  c1_W_ref, c1_b_ref,
                  g_ref, s_ref, de_ref, m1_ref):
    i = pl.program_id(0)

    @pl.when(i == 0)
    def _init():
        de_ref[...] = jnp.zeros_like(de_ref)
        m1_ref[...] = jnp.zeros_like(m1_ref)

    h_tile = h_ref[...]
    w_row = w_ref[...]

    dv = jnp.sum(h_tile * w_row, axis=1, keepdims=True)
    s = jax.lax.rsqrt(dv + EPS)
    s_ref[...] = s
    de_ref[...] += jnp.sum(h_tile, axis=0, keepdims=True)

    x1 = x_ref[...] @ psi_W_ref[...] + psi_b_ref[...]
    z1 = z_ref[...] @ phi_W_ref[...] + phi_b_ref[...]
    cat = jnp.concatenate([x1, z1], axis=1)
    gh = jnp.maximum(cat @ g1_W_ref[...] + g1_b_ref[...], 0.0)
    g = jax.nn.sigmoid(gh @ g2_W_ref[...] + g2_b_ref[...])
    g_ref[...] = g
    fused = g * z1 + (1.0 - g) * x1

    x1c = fused @ c1_W_ref[...] + c1_b_ref[...]
    xn1 = x1c * s
    m1_ref[...] += jax.lax.dot_general(
        h_tile, xn1, (((0,), (0,)), ((), ())),
        preferred_element_type=jnp.float32)


def _pass2_kernel(h_ref, s_ref, m1_ref, w_ref, de_ref,
                  c2_W_ref, c2_b_ref, m2_ref):
    i = pl.program_id(0)

    @pl.when(i == 0)
    def _init():
        m2_ref[...] = jnp.zeros_like(m2_ref)

    h_tile = h_ref[...]
    s = s_ref[...]
    m1n = m1_ref[...] * (w_ref[...] / (de_ref[...] + EPS))
    y1 = jnp.dot(h_tile, m1n, preferred_element_type=jnp.float32)
    h1 = jnp.maximum(y1 * s, 0.0)
    x2 = h1 @ c2_W_ref[...] + c2_b_ref[...]
    xn2 = x2 * s
    m2_ref[...] += jax.lax.dot_general(
        h_tile, xn2, (((0,), (0,)), ((), ())),
        preferred_element_type=jnp.float32)


def _pass3_kernel(h_ref, s_ref, m2_ref, w_ref, de_ref,
                  hd_W_ref, hd_b_ref, out_ref):
    h_tile = h_ref[...]
    s = s_ref[...]
    m2n = m2_ref[...] * (w_ref[...] / (de_ref[...] + EPS))
    y2 = jnp.dot(h_tile, m2n, preferred_element_type=jnp.float32)
    h2 = jnp.maximum(y2 * s, 0.0)
    out_ref[...] = h2 @ hd_W_ref[...] + hd_b_ref[...]


def _full(shape):
    nd = len(shape)
    return pl.BlockSpec(shape, lambda i: (0,) * nd)


def kernel(x, z, H, w,
           psi_W, psi_b, phi_W, phi_b,
           g1_W, g1_b, g2_W, g2_b,
           c1_W, c1_b, c2_W, c2_b,
           hd_W, hd_b):
    grid = (N // BN,)
    w_row = w.reshape(1, M)
    w_col = w.reshape(M, 1)
    hb = H.shape[0]

    params = dict(
        grid=grid,
        compiler_params=pltpu.CompilerParams(
            dimension_semantics=("arbitrary",)),
    )

    row = lambda shape: pl.BlockSpec(shape, lambda i: (i, 0))

    g, s, de, m1 = pl.pallas_call(
        _pass1_kernel,
        in_specs=[row((BN, M)), row((BN, x.shape[1])), row((BN, z.shape[1])),
                  _full((1, M)),
                  _full(psi_W.shape), _full((1, psi_b.shape[0])),
                  _full(phi_W.shape), _full((1, phi_b.shape[0])),
                  _full(g1_W.shape), _full((1, g1_b.shape[0])),
                  _full(g2_W.shape), _full((1, g2_b.shape[0])),
                  _full(c1_W.shape), _full((1, c1_b.shape[0]))],
        out_specs=[row((BN, 32)), row((BN, 1)), _full((1, M)), _full((M, 64))],
        out_shape=[jax.ShapeDtypeStruct((N, 32), jnp.float32),
                   jax.ShapeDtypeStruct((N, 1), jnp.float32),
                   jax.ShapeDtypeStruct((1, M), jnp.float32),
                   jax.ShapeDtypeStruct((M, 64), jnp.float32)],
        **params,
    )(H, x, z, w_row,
      psi_W, psi_b.reshape(1, -1), phi_W, phi_b.reshape(1, -1),
      g1_W, g1_b.reshape(1, -1), g2_W, g2_b.reshape(1, -1),
      c1_W, c1_b.reshape(1, -1))

    de_col = de.reshape(M, 1)

    m2 = pl.pallas_call(
        _pass2_kernel,
        in_specs=[row((BN, M)), row((BN, 1)), _full((M, 64)),
                  _full((M, 1)), _full((M, 1)),
                  _full(c2_W.shape), _full((1, c2_b.shape[0]))],
        out_specs=_full((M, 64)),
        out_shape=jax.ShapeDtypeStruct((M, 64), jnp.float32),
        **params,
    )(H, s, m1, w_col, de_col, c2_W, c2_b.reshape(1, -1))

    logits = pl.pallas_call(
        _pass3_kernel,
        in_specs=[row((BN, M)), row((BN, 1)), _full((M, 64)),
                  _full((M, 1)), _full((M, 1)),
                  _full(hd_W.shape), _full((1, hd_b.shape[0]))],
        out_specs=row((BN, hd_b.shape[0])),
        out_shape=jax.ShapeDtypeStruct((N, hd_b.shape[0]), jnp.float32),
        **params,
    )(H, s, m2, w_col, de_col, hd_W, hd_b.reshape(1, -1))

    return (logits, g)
